# baseline (speedup 1.0000x reference)
import functools

import jax
import jax.numpy as jnp
from jax import lax
from jax.experimental import pallas as pl
from jax.experimental.pallas import tpu as pltpu
from jax.experimental.pallas import tpu_sc as plsc

NC = 2
NS = 16
NW = NC * NS
L = 16

BATCH = 16384
K_DIM = 32
B_PER_W = BATCH // NW
CHUNK = 128
N_CHUNK = B_PER_W // CHUNK
ROWS128 = BATCH // CHUNK


def _sc_gather_partials(users2d, items2d, ue, ie, ub_flat, ib_flat):
  mesh = plsc.VectorSubcoreMesh(
      core_axis_name="c", subcore_axis_name="s",
      num_cores=NC, num_subcores=NS)

  @functools.partial(
      pl.kernel,
      out_type=(
          jax.ShapeDtypeStruct((NW, 2 * L), jnp.float32),
          jax.ShapeDtypeStruct((ROWS128, CHUNK), jnp.float32),
          jax.ShapeDtypeStruct((ROWS128, CHUNK), jnp.float32),
      ),
      mesh=mesh,
      compiler_params=pltpu.CompilerParams(use_tc_tiling_on_sc=False),
      scratch_types=[
          pltpu.VMEM((N_CHUNK, CHUNK), jnp.int32),
          pltpu.VMEM((N_CHUNK, CHUNK), jnp.int32),
          pltpu.VMEM((B_PER_W, K_DIM), jnp.float32),
          pltpu.VMEM((B_PER_W, K_DIM), jnp.float32),
          pltpu.VMEM((N_CHUNK, CHUNK), jnp.float32),
          pltpu.VMEM((N_CHUNK, CHUNK), jnp.float32),
          pltpu.VMEM((2 * L,), jnp.float32),
          pltpu.SemaphoreType.DMA,
          pltpu.SemaphoreType.DMA,
      ],
  )
  def k(u_hbm, i_hbm, ue_hbm, ie_hbm, ub_hbm, ib_hbm,
        part_out, ubg_out, ibg_out,
        idx_u, idx_i, rows_u, rows_i, bu, bi, part_v, sem_e, sem_b):
    wid = lax.axis_index("s") * NC + lax.axis_index("c")
    r0 = wid * N_CHUNK
    pltpu.sync_copy(u_hbm.at[pl.ds(r0, N_CHUNK)], idx_u)
    pltpu.sync_copy(i_hbm.at[pl.ds(r0, N_CHUNK)], idx_i)
    handles = []
    for j in range(N_CHUNK):
      handles.append(pltpu.async_copy(
          ue_hbm.at[idx_u.at[j]], rows_u.at[pl.ds(j * CHUNK, CHUNK)], sem_e))
      handles.append(pltpu.async_copy(
          ie_hbm.at[idx_i.at[j]], rows_i.at[pl.ds(j * CHUNK, CHUNK)], sem_e))
      handles.append(pltpu.async_copy(
          ub_hbm.at[idx_u.at[j]], bu.at[j], sem_b))
      handles.append(pltpu.async_copy(
          ib_hbm.at[idx_i.at[j]], bi.at[j], sem_b))
    for h in handles:
      h.wait()

    pltpu.sync_copy(bu, ubg_out.at[pl.ds(r0, N_CHUNK)])
    pltpu.sync_copy(bi, ibg_out.at[pl.ds(r0, N_CHUNK)])
    l2b = jnp.zeros((L,), jnp.float32)
    for j in range(N_CHUNK):
      for c in range(CHUNK // L):
        x = bu[j, pl.ds(c * L, L)]
        y = bi[j, pl.ds(c * L, L)]
        l2b = l2b + x * x + y * y

    def body(i, carry):
      dot, l2 = carry
      u0 = rows_u[i, pl.ds(0, L)]
      u1 = rows_u[i, pl.ds(L, L)]
      v0 = rows_i[i, pl.ds(0, L)]
      v1 = rows_i[i, pl.ds(L, L)]
      dot = dot + u0 * v0 + u1 * v1
      l2 = l2 + u0 * u0 + u1 * u1 + v0 * v0 + v1 * v1
      return dot, l2

    dot, l2 = lax.fori_loop(
        0, B_PER_W, body,
        (jnp.zeros((L,), jnp.float32), l2b))
    part_v[pl.ds(0, L)] = dot
    part_v[pl.ds(L, L)] = l2
    pltpu.sync_copy(part_v, part_out.at[wid])

  return k(users2d, items2d, ue, ie, ub_flat, ib_flat)


def _tc_finish(part, ubg, ibg, gb):
  def body(part_ref, ubg_ref, ibg_ref, gb_ref, out_ref, l2_ref):
    p = part_ref[...]
    dot_all = jnp.sum(p[:, 0:L])
    l2 = jnp.sum(p[:, L:2 * L])
    z = dot_all + gb_ref[0, 0] + ubg_ref[...] + ibg_ref[...]
    out_ref[...] = jax.nn.sigmoid(z)
    l2_ref[0, 0] = l2

  return pl.pallas_call(
      body,
      out_shape=(
          jax.ShapeDtypeStruct((ROWS128, CHUNK), jnp.float32),
          jax.ShapeDtypeStruct((1, 1), jnp.float32),
      ),
      in_specs=[
          pl.BlockSpec(memory_space=pltpu.VMEM),
          pl.BlockSpec(memory_space=pltpu.VMEM),
          pl.BlockSpec(memory_space=pltpu.VMEM),
          pl.BlockSpec(memory_space=pltpu.SMEM),
      ],
      out_specs=(
          pl.BlockSpec(memory_space=pltpu.VMEM),
          pl.BlockSpec(memory_space=pltpu.SMEM),
      ),
  )(part, ubg, ibg, gb)


def kernel(users, items, users_embedding, users_bias, items_embedding,
           items_bias, global_bias):
  users2d = users.astype(jnp.int32).reshape(ROWS128, CHUNK)
  items2d = items.astype(jnp.int32).reshape(ROWS128, CHUNK)
  ub_flat = users_bias.reshape(-1)
  ib_flat = items_bias.reshape(-1)
  part, ubg, ibg = _sc_gather_partials(
      users2d, items2d, users_embedding, items_embedding, ub_flat, ib_flat)
  out2d, l2 = _tc_finish(part, ubg, ibg, global_bias)
  return out2d.reshape(BATCH, 1), l2[0, 0]
